# Initial kernel scaffold; baseline (speedup 1.0000x reference)
import jax
import jax.numpy as jnp
from jax.experimental import pallas as pl


def kernel(x, edge_index, W0, W1):
    raise NotImplementedError("write your pallas kernel here")


# speedup vs baseline: 7.5765x; 7.5765x over previous
import functools

import jax
import jax.numpy as jnp
from jax import lax
from jax.experimental import pallas as pl
from jax.experimental.pallas import tpu as pltpu
from jax.experimental.pallas import tpu_sc as plsc

N = 10000
D = 128
E = 320000

NC = 2
NS = 16
NW = NC * NS
EPT = E // NW
CH = 128
NFULL = EPT // CH
REM = EPT - NFULL * CH
NP = 10240
ROWS_PT = NP // NS


def _seg_body(x_hbm, src_hbm, dst_hbm, out0, out1, acc, srcv, dstv, rows,
              dstr, rowsr, sem):
    c = lax.axis_index("c")
    s = lax.axis_index("s")
    wid = c * NS + s
    ebase = wid * EPT

    def _zrow(r, carry):
        for jc in range(D // 16):
            rows[r, pl.ds(jc * 16, 16)] = jnp.zeros((16,), jnp.float32)
        return carry

    lax.fori_loop(0, CH, _zrow, 0)
    row0 = pl.multiple_of(s * ROWS_PT, 8)
    for off in range(0, ROWS_PT, CH):
        pltpu.sync_copy(rows, acc.at[pl.ds(row0 + off, CH)])
    plsc.subcore_barrier()

    pltpu.sync_copy(src_hbm.at[pl.ds(ebase, EPT)], srcv)

    def _chunk(j, carry):
        eb = ebase + j * CH
        pltpu.sync_copy(dst_hbm.at[pl.ds(eb, CH)], dstv)
        pltpu.async_copy(x_hbm.at[srcv.at[pl.ds(j * CH, CH)]], rows, sem).wait()
        pltpu.sync_copy(rows, acc.at[dstv], add=True)
        return carry

    lax.fori_loop(0, NFULL, _chunk, 0)
    if REM:
        eb = ebase + NFULL * CH
        pltpu.sync_copy(dst_hbm.at[pl.ds(eb, REM)], dstr)
        pltpu.async_copy(x_hbm.at[srcv.at[pl.ds(NFULL * CH, REM)]], rowsr,
                         sem).wait()
        pltpu.sync_copy(rowsr, acc.at[dstr], add=True)
    plsc.subcore_barrier()

    @pl.when(c == 0)
    def _():
        pltpu.sync_copy(acc.at[pl.ds(row0, ROWS_PT)],
                        out0.at[pl.ds(row0, ROWS_PT)])

    @pl.when(c == 1)
    def _():
        pltpu.sync_copy(acc.at[pl.ds(row0, ROWS_PT)],
                        out1.at[pl.ds(row0, ROWS_PT)])


_segsum = pl.kernel(
    _seg_body,
    out_type=(jax.ShapeDtypeStruct((NP, D), jnp.float32),
              jax.ShapeDtypeStruct((NP, D), jnp.float32)),
    mesh=plsc.VectorSubcoreMesh(core_axis_name="c", subcore_axis_name="s"),
    scratch_types=[
        pltpu.VMEM_SHARED((NP, D), jnp.float32),
        pltpu.VMEM((EPT,), jnp.int32),
        pltpu.VMEM((CH,), jnp.int32),
        pltpu.VMEM((CH, D), jnp.float32),
        pltpu.VMEM((REM,), jnp.int32),
        pltpu.VMEM((REM, D), jnp.float32),
        pltpu.SemaphoreType.DMA,
    ],
)


def _mm_body(a_ref, b_ref, o_ref):
    o_ref[...] = jnp.dot(a_ref[...], b_ref[...],
                         preferred_element_type=jnp.float32)


_mm = pl.pallas_call(
    _mm_body,
    out_shape=jax.ShapeDtypeStruct((D, D), jnp.float32),
)

_BR = 2000


def _add_body(a_ref, b_ref, o_ref):
    o_ref[...] = a_ref[...] + b_ref[...]


_add = pl.pallas_call(
    _add_body,
    grid=(N // _BR,),
    in_specs=[pl.BlockSpec((_BR, D), lambda i: (i, 0)),
              pl.BlockSpec((_BR, D), lambda i: (i, 0))],
    out_specs=pl.BlockSpec((_BR, D), lambda i: (i, 0)),
    out_shape=jax.ShapeDtypeStruct((N, D), jnp.float32),
)


def _final_body(q0_ref, q1_ref, w_ref, o_ref):
    t = q0_ref[...] + q1_ref[...]
    y = jnp.dot(t, w_ref[...], preferred_element_type=jnp.float32)
    m = jnp.max(y, axis=-1, keepdims=True)
    lse = jnp.log(jnp.sum(jnp.exp(y - m), axis=-1, keepdims=True))
    o_ref[...] = y - m - lse


_final = pl.pallas_call(
    _final_body,
    grid=(N // _BR,),
    in_specs=[pl.BlockSpec((_BR, D), lambda i: (i, 0)),
              pl.BlockSpec((_BR, D), lambda i: (i, 0)),
              pl.BlockSpec((D, D), lambda i: (0, 0))],
    out_specs=pl.BlockSpec((_BR, D), lambda i: (i, 0)),
    out_shape=jax.ShapeDtypeStruct((N, D), jnp.float32),
)


def kernel(x, edge_index, W0, W1):
    src = edge_index[0].astype(jnp.int32)
    dst = edge_index[1].astype(jnp.int32)
    w = _mm(W0, W1)
    p0, p1 = _segsum(x, src, dst)
    t1 = _add(p0, p1)
    q0, q1 = _segsum(t1, src, dst)
    return _final(q0, q1, w)
